# baseline (speedup 1.0000x reference)
import functools

import jax
import jax.numpy as jnp
from jax import lax
from jax.experimental import pallas as pl
from jax.experimental.pallas import tpu as pltpu
from jax.experimental.pallas import tpu_sc as plsc

_NE, _ND, _NT = 100000, 500, 2000
_E = 100000
_DIN = 128
_HID = 64
_HEADS = 4
_DOUT = 16
_HP = 16
_EPAD = 106496
_B = 256
_BG = 64
_SMALLBITS = 11
_SMALLPAD = 2047
_BIGPAD = 131071
_SPMEM_WORDS = 2097151


def _ru(x, m):
    return -(-x // m) * m


def _mesh():
    return plsc.VectorSubcoreMesh(core_axis_name="c", subcore_axis_name="s")


def _zero_fill(buf, rows, width16):
    z = jnp.zeros((16,), jnp.float32)

    def zb(r, _):
        for k in range(width16):
            buf[r, pl.ds(k * 16, 16)] = z
        return 0
    lax.fori_loop(0, rows, zb, 0)


def _zero_shared(zer, zrows, acc, base, zt, sem):
    nz = -(-zt // zrows)
    cps = []
    for k in range(nz):
        st = min(k * zrows, zt - zrows)
        cps.append(pltpu.async_copy(zer.at[pl.ds(0, zrows)],
                                    acc.at[pl.ds(base + st, zrows)], sem))
    for cp in cps:
        cp.wait()


def _unpack(w, big_is_dst):
    small = w & _SMALLPAD
    big = w >> _SMALLBITS
    if big_is_dst:
        return small, big
    return big, small


@functools.lru_cache(None)
def _build_k1(D):
    big_dst = D > 2048
    Dh = _ru(D, 16) // 2
    JUNK = Dh
    DLr = _ru(Dh + 1, 128)
    ZT = DLr // 16
    ZB = min(128, ZT)
    EPT = _EPAD // 16
    NB = EPT // _B
    WT = _ru(-(-Dh // 16), 8)

    @functools.partial(
        pl.kernel,
        out_type=(jax.ShapeDtypeStruct((_EPAD, _HP), jnp.float32),
                  jax.ShapeDtypeStruct((2 * Dh, _HP), jnp.float32)),
        mesh=_mesh(),
        compiler_params=pltpu.CompilerParams(
            use_tc_tiling_on_sc=False, needs_layout_passes=False),
        scratch_types=[
            pltpu.VMEM((EPT,), jnp.int32),
            pltpu.VMEM((2, 128), jnp.int32),
            pltpu.VMEM((2, 128), jnp.int32),
            pltpu.VMEM((2, 128), jnp.int32),
            pltpu.VMEM((_B, _HP), jnp.float32),
            pltpu.VMEM((_B, _HP), jnp.float32),
            pltpu.VMEM((_B, _HP), jnp.float32),
            pltpu.VMEM((128, _HP), jnp.float32),
            pltpu.VMEM_SHARED((DLr, _HP), jnp.float32),
            pltpu.SemaphoreType.DMA,
            pltpu.SemaphoreType.DMA,
        ],
    )
    def k1(a_s, a_d, epk_in, s_out, den_out,
           epk, src2, dstc2, ldst2, as_r, ad_r, s_r, zer, den_sh,
           sem1, sem2):
        cid = lax.axis_index("c")
        sid = lax.axis_index("s")
        lo = cid * Dh

        pltpu.sync_copy(epk_in.at[pl.ds(sid * EPT, EPT)], epk)
        _zero_fill(zer, min(128, ZT), 1)
        _zero_shared(zer, min(128, ZT), den_sh, sid * ZT, ZT, sem1)
        plsc.subcore_barrier()

        def step(j, carry):
            off = sid * EPT + j * _B
            loff = j * _B
            for g in range(_B // 16):
                w = epk[pl.ds(loff + g * 16, 16)]
                sv, dv = _unpack(w, big_dst)
                dc = jnp.minimum(dv, D - 1)
                inh = (dv >= lo) & (dv < lo + Dh)
                ld = jnp.where(inh, dv - lo, JUNK)
                src2[g // 8, pl.ds((g % 8) * 16, 16)] = sv
                dstc2[g // 8, pl.ds((g % 8) * 16, 16)] = dc
                ldst2[g // 8, pl.ds((g % 8) * 16, 16)] = ld
            cps = []
            for k in range(2):
                cps.append(pltpu.async_copy(
                    a_s.at[src2.at[k]], as_r.at[pl.ds(k * 128, 128)], sem1))
                cps.append(pltpu.async_copy(
                    a_d.at[dstc2.at[k]], ad_r.at[pl.ds(k * 128, 128)], sem2))
            for cp in cps:
                cp.wait()

            def ebody(e4, _):
                for u in range(4):
                    e = e4 * 4 + u
                    x = as_r[e, :] + ad_r[e, :]
                    x = jnp.maximum(x, 0.2 * x)
                    s_r[e, :] = jnp.exp(x)
                return 0
            lax.fori_loop(0, _B // 4, ebody, 0)

            for k in range(2):
                pltpu.sync_copy(s_r.at[pl.ds(k * 128, 128)],
                                den_sh.at[ldst2.at[k]], add=True)

            @pl.when(cid == 0)
            def _():
                pltpu.sync_copy(s_r, s_out.at[pl.ds(off, _B)])
            return carry

        lax.fori_loop(0, NB, step, 0)
        plsc.subcore_barrier()
        a = jnp.minimum(sid * WT, Dh - WT)
        pltpu.sync_copy(den_sh.at[pl.ds(a, WT)], den_out.at[pl.ds(lo + a, WT)])

    return k1


@functools.lru_cache(None)
def _build_k2(D, F, nrels, fwd, H):
    C = F // H
    C16 = C // 16
    if fwd:
        EPW = _EPAD // 32
    else:
        EPW = _EPAD // 16
    NB = EPW // _B
    CAP = EPW + _BG
    ZR = 32

    tile_words = (nrels * EPW + 3 * CAP + 4 * _BG + _BG * F + 2 * _BG * _HP
                  + ZR * F)
    budget = _SPMEM_WORDS - 16 * tile_words - 16384
    if fwd:
        CH = _ru(D, 16)
        NCH = 1
        SH = None
    else:
        lim = (budget // F) // 128 * 128 - 128
        CH = 1 << (lim.bit_length() - 1)
        SH = CH.bit_length() - 1
        NCH = -(-D // CH)
    JUNK = CH
    CHr = _ru(CH + 1, 128)
    assert fwd or CHr * F <= budget, (CHr, F, budget)
    ZT = CHr // 16
    WT = _ru(-(-CH // 16), 8)

    if fwd:
        out_type = jax.ShapeDtypeStruct((2, CH, F), jnp.float32)
    else:
        out_type = jax.ShapeDtypeStruct((_ru(D, 16), F), jnp.float32)

    @functools.partial(
        pl.kernel,
        out_type=out_type,
        mesh=_mesh(),
        compiler_params=pltpu.CompilerParams(
            use_tc_tiling_on_sc=False, needs_layout_passes=False),
        scratch_types=[
            pltpu.VMEM((nrels, EPW), jnp.int32),
            pltpu.VMEM((CAP,), jnp.int32),
            pltpu.VMEM((CAP,), jnp.int32),
            pltpu.VMEM((CAP,), jnp.int32),
            pltpu.VMEM((_BG,), jnp.int32),
            pltpu.VMEM((_BG,), jnp.int32),
            pltpu.VMEM((_BG,), jnp.int32),
            pltpu.VMEM((_BG,), jnp.int32),
            pltpu.VMEM((_BG, F), jnp.float32),
            pltpu.VMEM((_BG, _HP), jnp.float32),
            pltpu.VMEM((_BG, _HP), jnp.float32),
            pltpu.VMEM((ZR, F), jnp.float32),
            pltpu.VMEM_SHARED((CHr, F), jnp.float32),
            pltpu.SemaphoreType.DMA,
            pltpu.SemaphoreType.DMA,
            pltpu.SemaphoreType.DMA,
        ],
    )
    def k2(*args):
        rel_refs = []
        for r in range(nrels):
            rel_refs.append(args[4 * r:4 * r + 4])
        out = args[4 * nrels]
        (epk, srcc, ldstc, eposc, src_bg, ldst_bg, epos_bg, dpos_bg,
         rows, srows, drows, zer, acc, sem1, sem2, sem3) = args[4 * nrels + 1:]
        cid = lax.axis_index("c")
        sid = lax.axis_index("s")

        _zero_fill(zer, ZR, F // 16)
        if fwd:
            ebase = (cid * 16 + sid) * EPW
        else:
            ebase = sid * EPW
        for r in range(nrels):
            pltpu.sync_copy(rel_refs[r][1].at[pl.ds(ebase, EPW)], epk.at[r])

        def chunk_body(lo, ck, valid):
            _zero_shared(zer, ZR, acc, sid * ZT, ZT, sem1)
            plsc.subcore_barrier()

            for r, (xs, ew, s_in, den) in enumerate(rel_refs):
                def scan(j, cnt):
                    off = ebase + j * _B
                    loff = j * _B
                    for g in range(_B // 16):
                        w = epk[r, pl.ds(loff + g * 16, 16)]
                        sv, dv = _unpack(w, not fwd)
                        if fwd:
                            m = (dv >= lo) & (dv < lo + CH)
                        else:
                            m = (dv >> SH) == ck
                        ld = dv - lo
                        ep = off + g * 16 + lax.iota(jnp.int32, 16)
                        mi = m.astype(jnp.int32)
                        cs = plsc.cumsum(mi)
                        pos = cnt + cs - mi
                        plsc.store_scatter(srcc, [pos], sv, mask=m)
                        plsc.store_scatter(ldstc, [pos], ld, mask=m)
                        plsc.store_scatter(eposc, [pos], ep, mask=m)
                        cnt = cnt + cs[15]
                    return cnt
                cnt = lax.fori_loop(0, NB, scan, 0)

                zi = jnp.zeros((16,), jnp.int32)
                ji = jnp.full((16,), JUNK, jnp.int32)
                iota16 = lax.iota(jnp.int32, 16)
                for k in range(_BG // 16):
                    pos = cnt + k * 16 + iota16
                    plsc.store_scatter(srcc, [pos], zi)
                    plsc.store_scatter(ldstc, [pos], ji)
                    plsc.store_scatter(eposc, [pos], zi)

                nbat = (cnt + _BG - 1) // _BG

                def proc(i, _):
                    o = i * _BG
                    for k in range(_BG // 16):
                        sv = srcc[pl.ds(o + k * 16, 16)]
                        src_bg[pl.ds(k * 16, 16)] = sv
                        lv = ldstc[pl.ds(o + k * 16, 16)]
                        ldst_bg[pl.ds(k * 16, 16)] = lv
                        dpos_bg[pl.ds(k * 16, 16)] = jnp.minimum(lv + lo, D - 1)
                        ev = eposc[pl.ds(o + k * 16, 16)]
                        epos_bg[pl.ds(k * 16, 16)] = ev
                    g1 = pltpu.async_copy(xs.at[src_bg], rows, sem1)
                    g2 = pltpu.async_copy(s_in.at[epos_bg], srows, sem2)
                    g3 = pltpu.async_copy(den.at[dpos_bg], drows, sem3)
                    g1.wait()
                    g2.wait()
                    g3.wait()

                    def scale(e, _):
                        sv2 = srows[e, :]
                        dv2 = drows[e, :]
                        av = sv2 / (dv2 + 1e-16)
                        for h in range(H):
                            a_h = av[h]
                            for k2_ in range(C16):
                                col = h * C + k2_ * 16
                                rows[e, pl.ds(col, 16)] = (
                                    rows[e, pl.ds(col, 16)] * a_h)
                        return 0
                    lax.fori_loop(0, _BG, scale, 0)
                    pltpu.sync_copy(rows, acc.at[ldst_bg], add=True)
                    return 0

                lax.fori_loop(0, nbat, proc, 0)

            plsc.subcore_barrier()
            a = jnp.maximum(0, jnp.minimum(sid * WT, valid - WT))
            if fwd:
                pltpu.sync_copy(acc.at[pl.ds(a, WT)],
                                out.at[cid, pl.ds(lo + a, WT)])
            else:
                pltpu.sync_copy(acc.at[pl.ds(a, WT)],
                                out.at[pl.ds(lo + a, WT)])
            plsc.subcore_barrier()

        if fwd:
            chunk_body(0, 0, CH)
        else:
            nch = jnp.where(cid == 0, (NCH + 1) // 2, NCH // 2)

            def cloop(k, _):
                ck = 2 * k + cid
                lo = ck * CH
                valid = jnp.minimum(CH, _ru(D, 16) - lo)
                chunk_body(lo, ck, valid)
                return 0
            lax.fori_loop(0, nch, cloop, 0)

    return k2


_BM = 1024


@functools.lru_cache(None)
def _build_tc_matmul(M, K, Ns, relu_in, bias_out, logsoftmax):
    nm = -(-M // _BM)
    N = sum(Ns)
    offs = []
    o = 0
    for n in Ns:
        offs.append(o)
        o += n

    in_specs = [pl.BlockSpec((_BM, K), lambda i: (i, 0)),
                pl.BlockSpec((K, N), lambda i: (0, 0))]
    if relu_in:
        in_specs.append(pl.BlockSpec((1, K), lambda i: (0, 0)))
    if bias_out:
        in_specs.append(pl.BlockSpec((1, N), lambda i: (0, 0)))
    out_specs = [pl.BlockSpec((_BM, n), lambda i: (i, 0)) for n in Ns]
    out_shape = [jax.ShapeDtypeStruct((M, n), jnp.float32) for n in Ns]

    def body(*refs):
        a_ref, b_ref = refs[0], refs[1]
        idx = 2
        x = a_ref[...]
        if relu_in:
            x = jnp.maximum(x + refs[idx][...], 0.0)
            idx += 1
        z = jnp.dot(x, b_ref[...], preferred_element_type=jnp.float32)
        if bias_out:
            z = z + refs[idx][...]
            idx += 1
        if logsoftmax:
            mx = jnp.max(z, axis=1, keepdims=True)
            ex = jnp.exp(z - mx)
            z = z - mx - jnp.log(jnp.sum(ex, axis=1, keepdims=True))
        outs = refs[-len(Ns):]
        for j, o_ref in enumerate(outs):
            o_ref[...] = z[:, offs[j]:offs[j] + Ns[j]]

    return pl.pallas_call(
        body,
        grid=(nm,),
        in_specs=in_specs,
        out_specs=out_specs,
        out_shape=out_shape,
    )


def _tc_matmul(A, Bs, bias_in=None, bias_out=None, logsoftmax=False):
    M, K = A.shape
    Ns = tuple(b.shape[1] for b in Bs)
    Bcat = jnp.concatenate(Bs, axis=1) if len(Bs) > 1 else Bs[0]
    args = [A, Bcat]
    if bias_in is not None:
        args.append(bias_in.reshape(1, K))
    if bias_out is not None:
        args.append(bias_out.reshape(1, sum(Ns)))
    f = _build_tc_matmul(M, K, Ns, bias_in is not None, bias_out is not None,
                         logsoftmax)
    outs = f(*args)
    return outs if len(Bs) > 1 else outs[0]


def _fold_att(W, att):
    din = W.shape[0]
    H, C = att.shape
    wt = (W.reshape(din, H, C) * att[None]).sum(-1)
    return jnp.pad(wt, ((0, 0), (0, _HP - H)))


def _pack_edges(src, dst, big_is_dst):
    src = src.astype(jnp.int32)
    dst = dst.astype(jnp.int32)
    if big_is_dst:
        w = (dst << _SMALLBITS) | src
        pad = _BIGPAD << _SMALLBITS
    else:
        w = (src << _SMALLBITS) | dst
        pad = _SMALLPAD
    return jnp.concatenate([w, jnp.full((_EPAD - _E,), pad, jnp.int32)])


def kernel(x_employee, x_department, x_title, src_works_in, dst_works_in,
           src_has_role, dst_has_role, src_rev_works_in, dst_rev_works_in,
           src_rev_has_role, dst_rev_has_role, params):
    x_e, x_d, x_t = x_employee, x_department, x_title
    p = params

    e_wi = _pack_edges(src_works_in, dst_works_in, False)
    e_hr = _pack_edges(src_has_role, dst_has_role, False)
    e_rwi = _pack_edges(src_rev_works_in, dst_rev_works_in, True)
    e_rhr = _pack_edges(src_rev_has_role, dst_rev_has_role, True)

    c1wi, c1hr, c1rwi, c1rhr = p['c1_wi'], p['c1_hr'], p['c1_rwi'], p['c1_rhr']
    c2rwi, c2rhr = p['c2_rwi'], p['c2_rhr']

    folds_e = jnp.concatenate([
        _fold_att(c1wi['W_src'], c1wi['att_src']),
        _fold_att(c1hr['W_src'], c1hr['att_src']),
        _fold_att(c1rwi['W_dst'], c1rwi['att_dst']),
        _fold_att(c1rhr['W_dst'], c1rhr['att_dst'])], axis=1)
    xs_wi, xs_hr, a_e = _tc_matmul(x_e, (
        c1wi['W_src'], c1hr['W_src'], folds_e))
    a_s_wi, a_s_hr = a_e[:, 0:16], a_e[:, 16:32]
    a_d_rwi, a_d_rhr = a_e[:, 32:48], a_e[:, 48:64]
    xs_rwi, a_d2 = _tc_matmul(x_d, (
        c1rwi['W_src'],
        jnp.concatenate([_fold_att(c1wi['W_dst'], c1wi['att_dst']),
                         _fold_att(c1rwi['W_src'], c1rwi['att_src'])], axis=1)))
    a_d_wi, a_s_rwi = a_d2[:, 0:16], a_d2[:, 16:32]
    xs_rhr, a_t2 = _tc_matmul(x_t, (
        c1rhr['W_src'],
        jnp.concatenate([_fold_att(c1hr['W_dst'], c1hr['att_dst']),
                         _fold_att(c1rhr['W_src'], c1rhr['att_src'])], axis=1)))
    a_d_hr, a_s_rhr = a_t2[:, 0:16], a_t2[:, 16:32]

    s_wi, den_wi = _build_k1(_ND)(a_s_wi, a_d_wi, e_wi)
    s_hr, den_hr = _build_k1(_NT)(a_s_hr, a_d_hr, e_hr)
    s_rwi, den_rwi = _build_k1(_NE)(a_s_rwi, a_d_rwi, e_rwi)
    s_rhr, den_rhr = _build_k1(_NE)(a_s_rhr, a_d_rhr, e_rhr)

    F1 = _HEADS * _HID
    outd = _build_k2(_ND, F1, 1, True, _HEADS)(xs_wi, e_wi, s_wi, den_wi)
    agg_d = outd[0, :_ND] + outd[1, :_ND]
    outt = _build_k2(_NT, F1, 1, True, _HEADS)(xs_hr, e_hr, s_hr, den_hr)
    agg_t = outt[0, :_NT] + outt[1, :_NT]
    agg_e = _build_k2(_NE, F1, 2, False, _HEADS)(
        xs_rwi, e_rwi, s_rwi, den_rwi, xs_rhr, e_rhr, s_rhr, den_rhr)[:_NE]

    a2_e = _tc_matmul(agg_e, (
        jnp.concatenate([_fold_att(c2rwi['W_dst'], c2rwi['att_dst']),
                         _fold_att(c2rhr['W_dst'], c2rhr['att_dst'])], axis=1),),
        bias_in=c1rwi['bias'] + c1rhr['bias'])
    a_d2_rwi, a_d2_rhr = a2_e[:, 0:16], a2_e[:, 16:32]
    xs2_rwi, a_s2_rwi = _tc_matmul(agg_d, (
        c2rwi['W_src'], _fold_att(c2rwi['W_src'], c2rwi['att_src'])),
        bias_in=c1wi['bias'])
    xs2_rhr, a_s2_rhr = _tc_matmul(agg_t, (
        c2rhr['W_src'], _fold_att(c2rhr['W_src'], c2rhr['att_src'])),
        bias_in=c1hr['bias'])

    s2_rwi, den2_rwi = _build_k1(_NE)(a_s2_rwi, a_d2_rwi, e_rwi)
    s2_rhr, den2_rhr = _build_k1(_NE)(a_s2_rhr, a_d2_rhr, e_rhr)
    agg2_e = _build_k2(_NE, _HID, 2, False, 1)(
        xs2_rwi, e_rwi, s2_rwi, den2_rwi,
        xs2_rhr, e_rhr, s2_rhr, den2_rhr)[:_NE]

    return _tc_matmul(agg2_e, (p['lin_W'],),
                      bias_in=c2rwi['bias'] + c2rhr['bias'],
                      bias_out=p['lin_b'], logsoftmax=True)


# speedup vs baseline: 1.0141x; 1.0141x over previous
import functools

import jax
import jax.numpy as jnp
from jax import lax
from jax.experimental import pallas as pl
from jax.experimental.pallas import tpu as pltpu
from jax.experimental.pallas import tpu_sc as plsc

_NE, _ND, _NT = 100000, 500, 2000
_E = 100000
_DIN = 128
_HID = 64
_HEADS = 4
_DOUT = 16
_HP = 16
_EPAD = 106496
_B = 256
_BG = 64
_SMALLBITS = 11
_SMALLPAD = 2047
_BIGPAD = 131071
_SPMEM_WORDS = 2097151


def _ru(x, m):
    return -(-x // m) * m


def _mesh():
    return plsc.VectorSubcoreMesh(core_axis_name="c", subcore_axis_name="s")


def _zero_fill(buf, rows, width16):
    z = jnp.zeros((16,), jnp.float32)

    def zb(r, _):
        for k in range(width16):
            buf[r, pl.ds(k * 16, 16)] = z
        return 0
    lax.fori_loop(0, rows, zb, 0)


def _zero_shared(zer, zrows, acc, base, zt, sem):
    nz = -(-zt // zrows)
    cps = []
    for k in range(nz):
        st = min(k * zrows, zt - zrows)
        cps.append(pltpu.async_copy(zer.at[pl.ds(0, zrows)],
                                    acc.at[pl.ds(base + st, zrows)], sem))
    for cp in cps:
        cp.wait()


def _unpack(w, big_is_dst):
    small = w & _SMALLPAD
    big = w >> _SMALLBITS
    if big_is_dst:
        return small, big
    return big, small


@functools.lru_cache(None)
def _build_k1(D):
    big_dst = D > 2048
    Dh = _ru(D, 16) // 2
    JUNK = Dh
    DLr = _ru(Dh + 1, 128)
    ZT = DLr // 16
    ZB = min(128, ZT)
    EPT = _EPAD // 16
    NB = EPT // _B
    WT = _ru(-(-Dh // 16), 8)

    @functools.partial(
        pl.kernel,
        out_type=(jax.ShapeDtypeStruct((_EPAD, _HP), jnp.float32),
                  jax.ShapeDtypeStruct((2 * Dh, _HP), jnp.float32)),
        mesh=_mesh(),
        compiler_params=pltpu.CompilerParams(
            use_tc_tiling_on_sc=False, needs_layout_passes=False),
        scratch_types=[
            pltpu.VMEM((EPT,), jnp.int32),
            pltpu.VMEM((4, 128), jnp.int32),
            pltpu.VMEM((4, 128), jnp.int32),
            pltpu.VMEM((4, 128), jnp.int32),
            pltpu.VMEM((2, _B, _HP), jnp.float32),
            pltpu.VMEM((2, _B, _HP), jnp.float32),
            pltpu.VMEM((_B, _HP), jnp.float32),
            pltpu.VMEM((128, _HP), jnp.float32),
            pltpu.VMEM_SHARED((DLr, _HP), jnp.float32),
            pltpu.SemaphoreType.DMA,
            pltpu.SemaphoreType.DMA,
            pltpu.SemaphoreType.DMA,
            pltpu.SemaphoreType.DMA,
        ],
    )
    def k1(a_s, a_d, epk_in, s_out, den_out,
           epk, src2, dstc2, ldst2, as_r, ad_r, s_r, zer, den_sh,
           sem_s0, sem_s1, sem_d0, sem_d1):
        cid = lax.axis_index("c")
        sid = lax.axis_index("s")
        lo = cid * Dh

        pltpu.sync_copy(epk_in.at[pl.ds(sid * EPT, EPT)], epk)
        _zero_fill(zer, min(128, ZT), 1)
        _zero_shared(zer, min(128, ZT), den_sh, sid * ZT, ZT, sem_s0)
        plsc.subcore_barrier()

        gsems = [(sem_s0, sem_d0), (sem_s1, sem_d1)]

        def stage_fire(j, bb):
            loff = j * _B
            for g in range(_B // 16):
                w = epk[pl.ds(loff + g * 16, 16)]
                sv, dv = _unpack(w, big_dst)
                dc = jnp.minimum(dv, D - 1)
                inh = (dv >= lo) & (dv < lo + Dh)
                ld = jnp.where(inh, dv - lo, JUNK)
                src2[2 * bb + g // 8, pl.ds((g % 8) * 16, 16)] = sv
                dstc2[2 * bb + g // 8, pl.ds((g % 8) * 16, 16)] = dc
                ldst2[2 * bb + g // 8, pl.ds((g % 8) * 16, 16)] = ld
            sa, sd = gsems[bb]
            for k in range(2):
                pltpu.async_copy(a_s.at[src2.at[2 * bb + k]],
                                 as_r.at[bb, pl.ds(k * 128, 128)], sa)
                pltpu.async_copy(a_d.at[dstc2.at[2 * bb + k]],
                                 ad_r.at[bb, pl.ds(k * 128, 128)], sd)

        def work(j, bb):
            sa, sd = gsems[bb]
            for k in range(2):
                pltpu.make_async_copy(
                    a_s.at[src2.at[2 * bb + k]],
                    as_r.at[bb, pl.ds(k * 128, 128)], sa).wait()
                pltpu.make_async_copy(
                    a_d.at[dstc2.at[2 * bb + k]],
                    ad_r.at[bb, pl.ds(k * 128, 128)], sd).wait()

            def ebody(e4, _):
                for u in range(4):
                    e = e4 * 4 + u
                    x = as_r[bb, e, :] + ad_r[bb, e, :]
                    x = jnp.maximum(x, 0.2 * x)
                    s_r[e, :] = jnp.exp(x)
                return 0
            lax.fori_loop(0, _B // 4, ebody, 0)

            for k in range(2):
                pltpu.sync_copy(s_r.at[pl.ds(k * 128, 128)],
                                den_sh.at[ldst2.at[2 * bb + k]], add=True)

            off = sid * EPT + j * _B

            @pl.when(cid == 0)
            def _():
                pltpu.sync_copy(s_r, s_out.at[pl.ds(off, _B)])

        stage_fire(0, 0)

        def step(j, carry):
            b = lax.rem(j, 2)
            for bb in range(2):
                @pl.when(b == bb)
                def _(bb=bb):
                    @pl.when(j + 1 < NB)
                    def _():
                        stage_fire(j + 1, 1 - bb)
                    work(j, bb)
            return carry

        lax.fori_loop(0, NB, step, 0)
        plsc.subcore_barrier()
        a = jnp.minimum(sid * WT, Dh - WT)
        pltpu.sync_copy(den_sh.at[pl.ds(a, WT)], den_out.at[pl.ds(lo + a, WT)])

    return k1


@functools.lru_cache(None)
def _build_k2(D, F, nrels, fwd, H):
    C = F // H
    C16 = C // 16
    if fwd:
        EPW = _EPAD // 32
    else:
        EPW = _EPAD // 16
    NB = EPW // _B
    CAP = EPW + _BG
    ZR = 32

    tile_words = (nrels * EPW + 3 * CAP + 4 * _BG + _BG * F + 2 * _BG * _HP
                  + ZR * F)
    budget = _SPMEM_WORDS - 16 * tile_words - 16384
    if fwd:
        CH = _ru(D, 16)
        NCH = 1
        SH = None
    else:
        lim = (budget // F) // 128 * 128 - 128
        CH = 1 << (lim.bit_length() - 1)
        SH = CH.bit_length() - 1
        NCH = -(-D // CH)
    JUNK = CH
    CHr = _ru(CH + 1, 128)
    assert fwd or CHr * F <= budget, (CHr, F, budget)
    ZT = CHr // 16
    WT = _ru(-(-CH // 16), 8)

    if fwd:
        out_type = jax.ShapeDtypeStruct((2, CH, F), jnp.float32)
    else:
        out_type = jax.ShapeDtypeStruct((_ru(D, 16), F), jnp.float32)

    @functools.partial(
        pl.kernel,
        out_type=out_type,
        mesh=_mesh(),
        compiler_params=pltpu.CompilerParams(
            use_tc_tiling_on_sc=False, needs_layout_passes=False),
        scratch_types=[
            pltpu.VMEM((nrels, EPW), jnp.int32),
            pltpu.VMEM((CAP,), jnp.int32),
            pltpu.VMEM((CAP,), jnp.int32),
            pltpu.VMEM((CAP,), jnp.int32),
            pltpu.VMEM((_BG,), jnp.int32),
            pltpu.VMEM((_BG,), jnp.int32),
            pltpu.VMEM((_BG,), jnp.int32),
            pltpu.VMEM((_BG,), jnp.int32),
            pltpu.VMEM((_BG, F), jnp.float32),
            pltpu.VMEM((_BG, _HP), jnp.float32),
            pltpu.VMEM((_BG, _HP), jnp.float32),
            pltpu.VMEM((ZR, F), jnp.float32),
            pltpu.VMEM_SHARED((CHr, F), jnp.float32),
            pltpu.SemaphoreType.DMA,
            pltpu.SemaphoreType.DMA,
            pltpu.SemaphoreType.DMA,
        ],
    )
    def k2(*args):
        rel_refs = []
        for r in range(nrels):
            rel_refs.append(args[4 * r:4 * r + 4])
        out = args[4 * nrels]
        (epk, srcc, ldstc, eposc, src_bg, ldst_bg, epos_bg, dpos_bg,
         rows, srows, drows, zer, acc, sem1, sem2, sem3) = args[4 * nrels + 1:]
        cid = lax.axis_index("c")
        sid = lax.axis_index("s")

        _zero_fill(zer, ZR, F // 16)
        if fwd:
            ebase = (cid * 16 + sid) * EPW
        else:
            ebase = sid * EPW
        for r in range(nrels):
            pltpu.sync_copy(rel_refs[r][1].at[pl.ds(ebase, EPW)], epk.at[r])

        def chunk_body(lo, ck, valid):
            _zero_shared(zer, ZR, acc, sid * ZT, ZT, sem1)
            plsc.subcore_barrier()

            for r, (xs, ew, s_in, den) in enumerate(rel_refs):
                def scan(j, cnt):
                    off = ebase + j * _B
                    loff = j * _B
                    for g in range(_B // 16):
                        w = epk[r, pl.ds(loff + g * 16, 16)]
                        sv, dv = _unpack(w, not fwd)
                        if fwd:
                            m = (dv >= lo) & (dv < lo + CH)
                        else:
                            m = (dv >> SH) == ck
                        ld = dv - lo
                        ep = off + g * 16 + lax.iota(jnp.int32, 16)
                        mi = m.astype(jnp.int32)
                        cs = plsc.cumsum(mi)
                        pos = cnt + cs - mi
                        plsc.store_scatter(srcc, [pos], sv, mask=m)
                        plsc.store_scatter(ldstc, [pos], ld, mask=m)
                        plsc.store_scatter(eposc, [pos], ep, mask=m)
                        cnt = cnt + cs[15]
                    return cnt
                cnt = lax.fori_loop(0, NB, scan, 0)

                zi = jnp.zeros((16,), jnp.int32)
                ji = jnp.full((16,), JUNK, jnp.int32)
                iota16 = lax.iota(jnp.int32, 16)
                for k in range(_BG // 16):
                    pos = cnt + k * 16 + iota16
                    plsc.store_scatter(srcc, [pos], zi)
                    plsc.store_scatter(ldstc, [pos], ji)
                    plsc.store_scatter(eposc, [pos], zi)

                nbat = (cnt + _BG - 1) // _BG

                def proc(i, _):
                    o = i * _BG
                    for k in range(_BG // 16):
                        sv = srcc[pl.ds(o + k * 16, 16)]
                        src_bg[pl.ds(k * 16, 16)] = sv
                        lv = ldstc[pl.ds(o + k * 16, 16)]
                        ldst_bg[pl.ds(k * 16, 16)] = lv
                        dpos_bg[pl.ds(k * 16, 16)] = jnp.minimum(lv + lo, D - 1)
                        ev = eposc[pl.ds(o + k * 16, 16)]
                        epos_bg[pl.ds(k * 16, 16)] = ev
                    g1 = pltpu.async_copy(xs.at[src_bg], rows, sem1)
                    g2 = pltpu.async_copy(s_in.at[epos_bg], srows, sem2)
                    g3 = pltpu.async_copy(den.at[dpos_bg], drows, sem3)
                    g1.wait()
                    g2.wait()
                    g3.wait()

                    def scale(e, _):
                        sv2 = srows[e, :]
                        dv2 = drows[e, :]
                        av = sv2 / (dv2 + 1e-16)
                        for h in range(H):
                            a_h = av[h]
                            for k2_ in range(C16):
                                col = h * C + k2_ * 16
                                rows[e, pl.ds(col, 16)] = (
                                    rows[e, pl.ds(col, 16)] * a_h)
                        return 0
                    lax.fori_loop(0, _BG, scale, 0)
                    pltpu.sync_copy(rows, acc.at[ldst_bg], add=True)
                    return 0

                lax.fori_loop(0, nbat, proc, 0)

            plsc.subcore_barrier()
            a = jnp.maximum(0, jnp.minimum(sid * WT, valid - WT))
            if fwd:
                pltpu.sync_copy(acc.at[pl.ds(a, WT)],
                                out.at[cid, pl.ds(lo + a, WT)])
            else:
                pltpu.sync_copy(acc.at[pl.ds(a, WT)],
                                out.at[pl.ds(lo + a, WT)])
            plsc.subcore_barrier()

        if fwd:
            chunk_body(0, 0, CH)
        else:
            nch = jnp.where(cid == 0, (NCH + 1) // 2, NCH // 2)

            def cloop(k, _):
                ck = 2 * k + cid
                lo = ck * CH
                valid = jnp.minimum(CH, _ru(D, 16) - lo)
                chunk_body(lo, ck, valid)
                return 0
            lax.fori_loop(0, nch, cloop, 0)

    return k2


_BM = 1024


@functools.lru_cache(None)
def _build_tc_matmul(M, K, Ns, relu_in, bias_out, logsoftmax):
    nm = -(-M // _BM)
    N = sum(Ns)
    offs = []
    o = 0
    for n in Ns:
        offs.append(o)
        o += n

    in_specs = [pl.BlockSpec((_BM, K), lambda i: (i, 0)),
                pl.BlockSpec((K, N), lambda i: (0, 0))]
    if relu_in:
        in_specs.append(pl.BlockSpec((1, K), lambda i: (0, 0)))
    if bias_out:
        in_specs.append(pl.BlockSpec((1, N), lambda i: (0, 0)))
    out_specs = [pl.BlockSpec((_BM, n), lambda i: (i, 0)) for n in Ns]
    out_shape = [jax.ShapeDtypeStruct((M, n), jnp.float32) for n in Ns]

    def body(*refs):
        a_ref, b_ref = refs[0], refs[1]
        idx = 2
        x = a_ref[...]
        if relu_in:
            x = jnp.maximum(x + refs[idx][...], 0.0)
            idx += 1
        z = jnp.dot(x, b_ref[...], preferred_element_type=jnp.float32)
        if bias_out:
            z = z + refs[idx][...]
            idx += 1
        if logsoftmax:
            mx = jnp.max(z, axis=1, keepdims=True)
            ex = jnp.exp(z - mx)
            z = z - mx - jnp.log(jnp.sum(ex, axis=1, keepdims=True))
        outs = refs[-len(Ns):]
        for j, o_ref in enumerate(outs):
            o_ref[...] = z[:, offs[j]:offs[j] + Ns[j]]

    return pl.pallas_call(
        body,
        grid=(nm,),
        in_specs=in_specs,
        out_specs=out_specs,
        out_shape=out_shape,
    )


def _tc_matmul(A, Bs, bias_in=None, bias_out=None, logsoftmax=False):
    M, K = A.shape
    Ns = tuple(b.shape[1] for b in Bs)
    Bcat = jnp.concatenate(Bs, axis=1) if len(Bs) > 1 else Bs[0]
    args = [A, Bcat]
    if bias_in is not None:
        args.append(bias_in.reshape(1, K))
    if bias_out is not None:
        args.append(bias_out.reshape(1, sum(Ns)))
    f = _build_tc_matmul(M, K, Ns, bias_in is not None, bias_out is not None,
                         logsoftmax)
    outs = f(*args)
    return outs if len(Bs) > 1 else outs[0]


def _fold_att(W, att):
    din = W.shape[0]
    H, C = att.shape
    wt = (W.reshape(din, H, C) * att[None]).sum(-1)
    return jnp.pad(wt, ((0, 0), (0, _HP - H)))


def _pack_edges(src, dst, big_is_dst):
    src = src.astype(jnp.int32)
    dst = dst.astype(jnp.int32)
    if big_is_dst:
        w = (dst << _SMALLBITS) | src
        pad = _BIGPAD << _SMALLBITS
    else:
        w = (src << _SMALLBITS) | dst
        pad = _SMALLPAD
    return jnp.concatenate([w, jnp.full((_EPAD - _E,), pad, jnp.int32)])


def kernel(x_employee, x_department, x_title, src_works_in, dst_works_in,
           src_has_role, dst_has_role, src_rev_works_in, dst_rev_works_in,
           src_rev_has_role, dst_rev_has_role, params):
    x_e, x_d, x_t = x_employee, x_department, x_title
    p = params

    e_wi = _pack_edges(src_works_in, dst_works_in, False)
    e_hr = _pack_edges(src_has_role, dst_has_role, False)
    e_rwi = _pack_edges(src_rev_works_in, dst_rev_works_in, True)
    e_rhr = _pack_edges(src_rev_has_role, dst_rev_has_role, True)

    c1wi, c1hr, c1rwi, c1rhr = p['c1_wi'], p['c1_hr'], p['c1_rwi'], p['c1_rhr']
    c2rwi, c2rhr = p['c2_rwi'], p['c2_rhr']

    folds_e = jnp.concatenate([
        _fold_att(c1wi['W_src'], c1wi['att_src']),
        _fold_att(c1hr['W_src'], c1hr['att_src']),
        _fold_att(c1rwi['W_dst'], c1rwi['att_dst']),
        _fold_att(c1rhr['W_dst'], c1rhr['att_dst'])], axis=1)
    xs_wi, xs_hr, a_e = _tc_matmul(x_e, (
        c1wi['W_src'], c1hr['W_src'], folds_e))
    a_s_wi, a_s_hr = a_e[:, 0:16], a_e[:, 16:32]
    a_d_rwi, a_d_rhr = a_e[:, 32:48], a_e[:, 48:64]
    xs_rwi, a_d2 = _tc_matmul(x_d, (
        c1rwi['W_src'],
        jnp.concatenate([_fold_att(c1wi['W_dst'], c1wi['att_dst']),
                         _fold_att(c1rwi['W_src'], c1rwi['att_src'])], axis=1)))
    a_d_wi, a_s_rwi = a_d2[:, 0:16], a_d2[:, 16:32]
    xs_rhr, a_t2 = _tc_matmul(x_t, (
        c1rhr['W_src'],
        jnp.concatenate([_fold_att(c1hr['W_dst'], c1hr['att_dst']),
                         _fold_att(c1rhr['W_src'], c1rhr['att_src'])], axis=1)))
    a_d_hr, a_s_rhr = a_t2[:, 0:16], a_t2[:, 16:32]

    s_wi, den_wi = _build_k1(_ND)(a_s_wi, a_d_wi, e_wi)
    s_hr, den_hr = _build_k1(_NT)(a_s_hr, a_d_hr, e_hr)
    s_rwi, den_rwi = _build_k1(_NE)(a_s_rwi, a_d_rwi, e_rwi)
    s_rhr, den_rhr = _build_k1(_NE)(a_s_rhr, a_d_rhr, e_rhr)

    F1 = _HEADS * _HID
    outd = _build_k2(_ND, F1, 1, True, _HEADS)(xs_wi, e_wi, s_wi, den_wi)
    agg_d = outd[0, :_ND] + outd[1, :_ND]
    outt = _build_k2(_NT, F1, 1, True, _HEADS)(xs_hr, e_hr, s_hr, den_hr)
    agg_t = outt[0, :_NT] + outt[1, :_NT]
    agg_e = _build_k2(_NE, F1, 2, False, _HEADS)(
        xs_rwi, e_rwi, s_rwi, den_rwi, xs_rhr, e_rhr, s_rhr, den_rhr)[:_NE]

    a2_e = _tc_matmul(agg_e, (
        jnp.concatenate([_fold_att(c2rwi['W_dst'], c2rwi['att_dst']),
                         _fold_att(c2rhr['W_dst'], c2rhr['att_dst'])], axis=1),),
        bias_in=c1rwi['bias'] + c1rhr['bias'])
    a_d2_rwi, a_d2_rhr = a2_e[:, 0:16], a2_e[:, 16:32]
    xs2_rwi, a_s2_rwi = _tc_matmul(agg_d, (
        c2rwi['W_src'], _fold_att(c2rwi['W_src'], c2rwi['att_src'])),
        bias_in=c1wi['bias'])
    xs2_rhr, a_s2_rhr = _tc_matmul(agg_t, (
        c2rhr['W_src'], _fold_att(c2rhr['W_src'], c2rhr['att_src'])),
        bias_in=c1hr['bias'])

    s2_rwi, den2_rwi = _build_k1(_NE)(a_s2_rwi, a_d2_rwi, e_rwi)
    s2_rhr, den2_rhr = _build_k1(_NE)(a_s2_rhr, a_d2_rhr, e_rhr)
    agg2_e = _build_k2(_NE, _HID, 2, False, 1)(
        xs2_rwi, e_rwi, s2_rwi, den2_rwi,
        xs2_rhr, e_rhr, s2_rhr, den2_rhr)[:_NE]

    return _tc_matmul(agg2_e, (p['lin_W'],),
                      bias_in=c2rwi['bias'] + c2rhr['bias'],
                      bias_out=p['lin_b'], logsoftmax=True)


# speedup vs baseline: 1.0414x; 1.0269x over previous
import functools

import jax
import jax.numpy as jnp
from jax import lax
from jax.experimental import pallas as pl
from jax.experimental.pallas import tpu as pltpu
from jax.experimental.pallas import tpu_sc as plsc

_NE, _ND, _NT = 100000, 500, 2000
_E = 100000
_DIN = 128
_HID = 64
_HEADS = 4
_DOUT = 16
_HP = 16
_EPAD = 106496
_B = 256
_BG = 64
_SMALLBITS = 11
_SMALLPAD = 2047
_BIGPAD = 131071
_SPMEM_WORDS = 2097151


def _ru(x, m):
    return -(-x // m) * m


def _mesh():
    return plsc.VectorSubcoreMesh(core_axis_name="c", subcore_axis_name="s")


def _zero_fill(buf, rows, width16):
    z = jnp.zeros((16,), jnp.float32)

    def zb(r, _):
        for k in range(width16):
            buf[r, pl.ds(k * 16, 16)] = z
        return 0
    lax.fori_loop(0, rows, zb, 0)


def _zero_shared(zer, zrows, acc, base, zt, sem):
    nz = -(-zt // zrows)
    cps = []
    for k in range(nz):
        st = min(k * zrows, zt - zrows)
        cps.append(pltpu.async_copy(zer.at[pl.ds(0, zrows)],
                                    acc.at[pl.ds(base + st, zrows)], sem))
    for cp in cps:
        cp.wait()


def _unpack(w, big_is_dst):
    small = w & _SMALLPAD
    big = w >> _SMALLBITS
    if big_is_dst:
        return small, big
    return big, small


@functools.lru_cache(None)
def _build_k1(D):
    big_dst = D > 2048
    Dh = _ru(D, 16) // 2
    JUNK = Dh
    DLr = _ru(Dh + 1, 128)
    ZT = DLr // 16
    ZB = min(128, ZT)
    EPT = _EPAD // 16
    NB = EPT // _B
    WT = _ru(-(-Dh // 16), 8)

    @functools.partial(
        pl.kernel,
        out_type=(jax.ShapeDtypeStruct((_EPAD, _HP), jnp.float32),
                  jax.ShapeDtypeStruct((2 * Dh, _HP), jnp.float32)),
        mesh=_mesh(),
        compiler_params=pltpu.CompilerParams(
            use_tc_tiling_on_sc=False, needs_layout_passes=False),
        scratch_types=[
            pltpu.VMEM((EPT,), jnp.int32),
            pltpu.VMEM((4, 128), jnp.int32),
            pltpu.VMEM((4, 128), jnp.int32),
            pltpu.VMEM((4, 128), jnp.int32),
            pltpu.VMEM((2, _B, _HP), jnp.float32),
            pltpu.VMEM((2, _B, _HP), jnp.float32),
            pltpu.VMEM((_B, _HP), jnp.float32),
            pltpu.VMEM((128, _HP), jnp.float32),
            pltpu.VMEM_SHARED((DLr, _HP), jnp.float32),
            pltpu.SemaphoreType.DMA,
            pltpu.SemaphoreType.DMA,
            pltpu.SemaphoreType.DMA,
            pltpu.SemaphoreType.DMA,
        ],
    )
    def k1(a_s, a_d, epk_in, s_out, den_out,
           epk, src2, dstc2, ldst2, as_r, ad_r, s_r, zer, den_sh,
           sem_s0, sem_s1, sem_d0, sem_d1):
        cid = lax.axis_index("c")
        sid = lax.axis_index("s")
        lo = cid * Dh

        pltpu.sync_copy(epk_in.at[pl.ds(sid * EPT, EPT)], epk)
        _zero_fill(zer, min(128, ZT), 1)
        _zero_shared(zer, min(128, ZT), den_sh, sid * ZT, ZT, sem_s0)
        plsc.subcore_barrier()

        gsems = [(sem_s0, sem_d0), (sem_s1, sem_d1)]

        def stage_fire(j, bb):
            loff = j * _B
            for g in range(_B // 16):
                w = epk[pl.ds(loff + g * 16, 16)]
                sv, dv = _unpack(w, big_dst)
                dc = jnp.minimum(dv, D - 1)
                inh = (dv >= lo) & (dv < lo + Dh)
                ld = jnp.where(inh, dv - lo, JUNK)
                src2[2 * bb + g // 8, pl.ds((g % 8) * 16, 16)] = sv
                dstc2[2 * bb + g // 8, pl.ds((g % 8) * 16, 16)] = dc
                ldst2[2 * bb + g // 8, pl.ds((g % 8) * 16, 16)] = ld
            sa, sd = gsems[bb]
            for k in range(2):
                pltpu.async_copy(a_s.at[src2.at[2 * bb + k]],
                                 as_r.at[bb, pl.ds(k * 128, 128)], sa)
                pltpu.async_copy(a_d.at[dstc2.at[2 * bb + k]],
                                 ad_r.at[bb, pl.ds(k * 128, 128)], sd)

        def work(j, bb):
            sa, sd = gsems[bb]
            for k in range(2):
                pltpu.make_async_copy(
                    a_s.at[src2.at[2 * bb + k]],
                    as_r.at[bb, pl.ds(k * 128, 128)], sa).wait()
                pltpu.make_async_copy(
                    a_d.at[dstc2.at[2 * bb + k]],
                    ad_r.at[bb, pl.ds(k * 128, 128)], sd).wait()

            def ebody(e4, _):
                for u in range(4):
                    e = e4 * 4 + u
                    x = as_r[bb, e, :] + ad_r[bb, e, :]
                    x = jnp.maximum(x, 0.2 * x)
                    s_r[e, :] = jnp.exp(x)
                return 0
            lax.fori_loop(0, _B // 4, ebody, 0)

            for k in range(2):
                pltpu.sync_copy(s_r.at[pl.ds(k * 128, 128)],
                                den_sh.at[ldst2.at[2 * bb + k]], add=True)

            off = sid * EPT + j * _B

            @pl.when(cid == 0)
            def _():
                pltpu.sync_copy(s_r, s_out.at[pl.ds(off, _B)])

        stage_fire(0, 0)

        def step(j, carry):
            b = lax.rem(j, 2)
            for bb in range(2):
                @pl.when(b == bb)
                def _(bb=bb):
                    @pl.when(j + 1 < NB)
                    def _():
                        stage_fire(j + 1, 1 - bb)
                    work(j, bb)
            return carry

        lax.fori_loop(0, NB, step, 0)
        plsc.subcore_barrier()
        a = jnp.minimum(sid * WT, Dh - WT)
        pltpu.sync_copy(den_sh.at[pl.ds(a, WT)], den_out.at[pl.ds(lo + a, WT)])

    return k1


@functools.lru_cache(None)
def _build_k2(D, F, nrels, fwd, H):
    C = F // H
    C16 = C // 16
    if fwd:
        EPW = _EPAD // 32
    else:
        EPW = _EPAD // 16
    NB = EPW // _B
    CAP = EPW + _BG
    ZR = 32

    tile_words = (nrels * EPW + 3 * CAP + 4 * _BG + _BG * F + 2 * _BG * _HP
                  + ZR * F)
    budget = _SPMEM_WORDS - 16 * tile_words - 16384
    if fwd:
        CH = _ru(D, 16)
        NCH = 1
        SH = None
    else:
        CH = (budget // F) // 128 * 128 - 128
        NCH = -(-D // CH)
    JUNK = CH
    CHr = _ru(CH + 1, 128)
    assert fwd or CHr * F <= budget, (CHr, F, budget)
    ZT = CHr // 16
    WT = _ru(-(-CH // 16), 8)

    if fwd:
        out_type = jax.ShapeDtypeStruct((2, CH, F), jnp.float32)
    else:
        out_type = jax.ShapeDtypeStruct((_ru(D, 16), F), jnp.float32)

    @functools.partial(
        pl.kernel,
        out_type=out_type,
        mesh=_mesh(),
        compiler_params=pltpu.CompilerParams(
            use_tc_tiling_on_sc=False, needs_layout_passes=False),
        scratch_types=[
            pltpu.VMEM((nrels, EPW), jnp.int32),
            pltpu.VMEM((CAP,), jnp.int32),
            pltpu.VMEM((CAP,), jnp.int32),
            pltpu.VMEM((CAP,), jnp.int32),
            pltpu.VMEM((_BG,), jnp.int32),
            pltpu.VMEM((_BG,), jnp.int32),
            pltpu.VMEM((_BG,), jnp.int32),
            pltpu.VMEM((_BG,), jnp.int32),
            pltpu.VMEM((_BG, F), jnp.float32),
            pltpu.VMEM((_BG, _HP), jnp.float32),
            pltpu.VMEM((_BG, _HP), jnp.float32),
            pltpu.VMEM((ZR, F), jnp.float32),
            pltpu.VMEM_SHARED((CHr, F), jnp.float32),
            pltpu.SemaphoreType.DMA,
            pltpu.SemaphoreType.DMA,
            pltpu.SemaphoreType.DMA,
        ],
    )
    def k2(*args):
        rel_refs = []
        for r in range(nrels):
            rel_refs.append(args[4 * r:4 * r + 4])
        out = args[4 * nrels]
        (epk, srcc, ldstc, eposc, src_bg, ldst_bg, epos_bg, dpos_bg,
         rows, srows, drows, zer, acc, sem1, sem2, sem3) = args[4 * nrels + 1:]
        cid = lax.axis_index("c")
        sid = lax.axis_index("s")

        _zero_fill(zer, ZR, F // 16)
        if fwd:
            ebase = (cid * 16 + sid) * EPW
        else:
            ebase = sid * EPW
        for r in range(nrels):
            pltpu.sync_copy(rel_refs[r][1].at[pl.ds(ebase, EPW)], epk.at[r])

        def chunk_body(lo, ck, valid):
            _zero_shared(zer, ZR, acc, sid * ZT, ZT, sem1)
            plsc.subcore_barrier()

            for r, (xs, ew, s_in, den) in enumerate(rel_refs):
                def scan(j, cnt):
                    off = ebase + j * _B
                    loff = j * _B
                    for g in range(_B // 16):
                        w = epk[r, pl.ds(loff + g * 16, 16)]
                        sv, dv = _unpack(w, not fwd)
                        m = (dv >= lo) & (dv < lo + CH)
                        ld = dv - lo
                        ep = off + g * 16 + lax.iota(jnp.int32, 16)
                        mi = m.astype(jnp.int32)
                        cs = plsc.cumsum(mi)
                        pos = cnt + cs - mi
                        plsc.store_scatter(srcc, [pos], sv, mask=m)
                        plsc.store_scatter(ldstc, [pos], ld, mask=m)
                        plsc.store_scatter(eposc, [pos], ep, mask=m)
                        cnt = cnt + cs[15]
                    return cnt
                cnt = lax.fori_loop(0, NB, scan, 0)

                zi = jnp.zeros((16,), jnp.int32)
                ji = jnp.full((16,), JUNK, jnp.int32)
                iota16 = lax.iota(jnp.int32, 16)
                for k in range(_BG // 16):
                    pos = cnt + k * 16 + iota16
                    plsc.store_scatter(srcc, [pos], zi)
                    plsc.store_scatter(ldstc, [pos], ji)
                    plsc.store_scatter(eposc, [pos], zi)

                nbat = (cnt + _BG - 1) // _BG

                def proc(i, _):
                    o = i * _BG
                    for k in range(_BG // 16):
                        sv = srcc[pl.ds(o + k * 16, 16)]
                        src_bg[pl.ds(k * 16, 16)] = sv
                        lv = ldstc[pl.ds(o + k * 16, 16)]
                        ldst_bg[pl.ds(k * 16, 16)] = lv
                        dpos_bg[pl.ds(k * 16, 16)] = jnp.minimum(lv + lo, D - 1)
                        ev = eposc[pl.ds(o + k * 16, 16)]
                        epos_bg[pl.ds(k * 16, 16)] = ev
                    g1 = pltpu.async_copy(xs.at[src_bg], rows, sem1)
                    g2 = pltpu.async_copy(s_in.at[epos_bg], srows, sem2)
                    g3 = pltpu.async_copy(den.at[dpos_bg], drows, sem3)
                    g1.wait()
                    g2.wait()
                    g3.wait()

                    def scale(e, _):
                        sv2 = srows[e, :]
                        dv2 = drows[e, :]
                        av = sv2 / (dv2 + 1e-16)
                        for h in range(H):
                            a_h = av[h]
                            for k2_ in range(C16):
                                col = h * C + k2_ * 16
                                rows[e, pl.ds(col, 16)] = (
                                    rows[e, pl.ds(col, 16)] * a_h)
                        return 0
                    lax.fori_loop(0, _BG, scale, 0)
                    pltpu.sync_copy(rows, acc.at[ldst_bg], add=True)
                    return 0

                lax.fori_loop(0, nbat, proc, 0)

            plsc.subcore_barrier()
            a = jnp.maximum(0, jnp.minimum(sid * WT, valid - WT))
            if fwd:
                pltpu.sync_copy(acc.at[pl.ds(a, WT)],
                                out.at[cid, pl.ds(lo + a, WT)])
            else:
                pltpu.sync_copy(acc.at[pl.ds(a, WT)],
                                out.at[pl.ds(lo + a, WT)])
            plsc.subcore_barrier()

        if fwd:
            chunk_body(0, 0, CH)
        else:
            nch = jnp.where(cid == 0, (NCH + 1) // 2, NCH // 2)

            def cloop(k, _):
                ck = 2 * k + cid
                lo = ck * CH
                valid = jnp.minimum(CH, _ru(D, 16) - lo)
                chunk_body(lo, ck, valid)
                return 0
            lax.fori_loop(0, nch, cloop, 0)

    return k2


_BM = 1024


@functools.lru_cache(None)
def _build_tc_matmul(M, K, Ns, relu_in, bias_out, logsoftmax):
    nm = -(-M // _BM)
    N = sum(Ns)
    offs = []
    o = 0
    for n in Ns:
        offs.append(o)
        o += n

    in_specs = [pl.BlockSpec((_BM, K), lambda i: (i, 0)),
                pl.BlockSpec((K, N), lambda i: (0, 0))]
    if relu_in:
        in_specs.append(pl.BlockSpec((1, K), lambda i: (0, 0)))
    if bias_out:
        in_specs.append(pl.BlockSpec((1, N), lambda i: (0, 0)))
    out_specs = [pl.BlockSpec((_BM, n), lambda i: (i, 0)) for n in Ns]
    out_shape = [jax.ShapeDtypeStruct((M, n), jnp.float32) for n in Ns]

    def body(*refs):
        a_ref, b_ref = refs[0], refs[1]
        idx = 2
        x = a_ref[...]
        if relu_in:
            x = jnp.maximum(x + refs[idx][...], 0.0)
            idx += 1
        z = jnp.dot(x, b_ref[...], preferred_element_type=jnp.float32)
        if bias_out:
            z = z + refs[idx][...]
            idx += 1
        if logsoftmax:
            mx = jnp.max(z, axis=1, keepdims=True)
            ex = jnp.exp(z - mx)
            z = z - mx - jnp.log(jnp.sum(ex, axis=1, keepdims=True))
        outs = refs[-len(Ns):]
        for j, o_ref in enumerate(outs):
            o_ref[...] = z[:, offs[j]:offs[j] + Ns[j]]

    return pl.pallas_call(
        body,
        grid=(nm,),
        in_specs=in_specs,
        out_specs=out_specs,
        out_shape=out_shape,
    )


def _tc_matmul(A, Bs, bias_in=None, bias_out=None, logsoftmax=False):
    M, K = A.shape
    Ns = tuple(b.shape[1] for b in Bs)
    Bcat = jnp.concatenate(Bs, axis=1) if len(Bs) > 1 else Bs[0]
    args = [A, Bcat]
    if bias_in is not None:
        args.append(bias_in.reshape(1, K))
    if bias_out is not None:
        args.append(bias_out.reshape(1, sum(Ns)))
    f = _build_tc_matmul(M, K, Ns, bias_in is not None, bias_out is not None,
                         logsoftmax)
    outs = f(*args)
    return outs if len(Bs) > 1 else outs[0]


def _fold_att(W, att):
    din = W.shape[0]
    H, C = att.shape
    wt = (W.reshape(din, H, C) * att[None]).sum(-1)
    return jnp.pad(wt, ((0, 0), (0, _HP - H)))


def _pack_edges(src, dst, big_is_dst):
    src = src.astype(jnp.int32)
    dst = dst.astype(jnp.int32)
    if big_is_dst:
        w = (dst << _SMALLBITS) | src
        pad = _BIGPAD << _SMALLBITS
    else:
        w = (src << _SMALLBITS) | dst
        pad = _SMALLPAD
    return jnp.concatenate([w, jnp.full((_EPAD - _E,), pad, jnp.int32)])


def kernel(x_employee, x_department, x_title, src_works_in, dst_works_in,
           src_has_role, dst_has_role, src_rev_works_in, dst_rev_works_in,
           src_rev_has_role, dst_rev_has_role, params):
    x_e, x_d, x_t = x_employee, x_department, x_title
    p = params

    e_wi = _pack_edges(src_works_in, dst_works_in, False)
    e_hr = _pack_edges(src_has_role, dst_has_role, False)
    e_rwi = _pack_edges(src_rev_works_in, dst_rev_works_in, True)
    e_rhr = _pack_edges(src_rev_has_role, dst_rev_has_role, True)

    c1wi, c1hr, c1rwi, c1rhr = p['c1_wi'], p['c1_hr'], p['c1_rwi'], p['c1_rhr']
    c2rwi, c2rhr = p['c2_rwi'], p['c2_rhr']

    folds_e = jnp.concatenate([
        _fold_att(c1wi['W_src'], c1wi['att_src']),
        _fold_att(c1hr['W_src'], c1hr['att_src']),
        _fold_att(c1rwi['W_dst'], c1rwi['att_dst']),
        _fold_att(c1rhr['W_dst'], c1rhr['att_dst'])], axis=1)
    xs_wi, xs_hr, a_e = _tc_matmul(x_e, (
        c1wi['W_src'], c1hr['W_src'], folds_e))
    a_s_wi, a_s_hr = a_e[:, 0:16], a_e[:, 16:32]
    a_d_rwi, a_d_rhr = a_e[:, 32:48], a_e[:, 48:64]
    xs_rwi, a_d2 = _tc_matmul(x_d, (
        c1rwi['W_src'],
        jnp.concatenate([_fold_att(c1wi['W_dst'], c1wi['att_dst']),
                         _fold_att(c1rwi['W_src'], c1rwi['att_src'])], axis=1)))
    a_d_wi, a_s_rwi = a_d2[:, 0:16], a_d2[:, 16:32]
    xs_rhr, a_t2 = _tc_matmul(x_t, (
        c1rhr['W_src'],
        jnp.concatenate([_fold_att(c1hr['W_dst'], c1hr['att_dst']),
                         _fold_att(c1rhr['W_src'], c1rhr['att_src'])], axis=1)))
    a_d_hr, a_s_rhr = a_t2[:, 0:16], a_t2[:, 16:32]

    s_wi, den_wi = _build_k1(_ND)(a_s_wi, a_d_wi, e_wi)
    s_hr, den_hr = _build_k1(_NT)(a_s_hr, a_d_hr, e_hr)
    s_rwi, den_rwi = _build_k1(_NE)(a_s_rwi, a_d_rwi, e_rwi)
    s_rhr, den_rhr = _build_k1(_NE)(a_s_rhr, a_d_rhr, e_rhr)

    F1 = _HEADS * _HID
    outd = _build_k2(_ND, F1, 1, True, _HEADS)(xs_wi, e_wi, s_wi, den_wi)
    agg_d = outd[0, :_ND] + outd[1, :_ND]
    outt = _build_k2(_NT, F1, 1, True, _HEADS)(xs_hr, e_hr, s_hr, den_hr)
    agg_t = outt[0, :_NT] + outt[1, :_NT]
    agg_e = _build_k2(_NE, F1, 2, False, _HEADS)(
        xs_rwi, e_rwi, s_rwi, den_rwi, xs_rhr, e_rhr, s_rhr, den_rhr)[:_NE]

    a2_e = _tc_matmul(agg_e, (
        jnp.concatenate([_fold_att(c2rwi['W_dst'], c2rwi['att_dst']),
                         _fold_att(c2rhr['W_dst'], c2rhr['att_dst'])], axis=1),),
        bias_in=c1rwi['bias'] + c1rhr['bias'])
    a_d2_rwi, a_d2_rhr = a2_e[:, 0:16], a2_e[:, 16:32]
    xs2_rwi, a_s2_rwi = _tc_matmul(agg_d, (
        c2rwi['W_src'], _fold_att(c2rwi['W_src'], c2rwi['att_src'])),
        bias_in=c1wi['bias'])
    xs2_rhr, a_s2_rhr = _tc_matmul(agg_t, (
        c2rhr['W_src'], _fold_att(c2rhr['W_src'], c2rhr['att_src'])),
        bias_in=c1hr['bias'])

    s2_rwi, den2_rwi = _build_k1(_NE)(a_s2_rwi, a_d2_rwi, e_rwi)
    s2_rhr, den2_rhr = _build_k1(_NE)(a_s2_rhr, a_d2_rhr, e_rhr)
    agg2_e = _build_k2(_NE, _HID, 2, False, 1)(
        xs2_rwi, e_rwi, s2_rwi, den2_rwi,
        xs2_rhr, e_rhr, s2_rhr, den2_rhr)[:_NE]

    return _tc_matmul(agg2_e, (p['lin_W'],),
                      bias_in=c2rwi['bias'] + c2rhr['bias'],
                      bias_out=p['lin_b'], logsoftmax=True)
